# baseline (speedup 1.0000x reference)
import jax
import jax.numpy as jnp
from jax.experimental import pallas as pl
from jax.experimental.pallas import tpu as pltpu

_SCHED = (1536, 13312, 1536)
_ARENA_ROWS = 14848
_X_OFF = tuple(sum(_SCHED[:i]) for i in range(len(_SCHED)))
_BUF_OFF = (0, 1536, 0)
_BUF_DEPS = {2: (0,)}


def _copy_body(x_ref, o_ref, arena, in_sems, out_sems):
    n = len(_SCHED)

    def in_cp(i):
        return pltpu.make_async_copy(
            x_ref.at[pl.ds(_X_OFF[i], _SCHED[i])],
            arena.at[pl.ds(_BUF_OFF[i], _SCHED[i])],
            in_sems.at[i],
        )

    def out_cp(i):
        return pltpu.make_async_copy(
            arena.at[pl.ds(_BUF_OFF[i], _SCHED[i])],
            o_ref.at[pl.ds(_X_OFF[i], _SCHED[i])],
            out_sems.at[i],
        )

    for j in range(n):
        if j not in _BUF_DEPS:
            in_cp(j).start()
    for i in range(n):
        in_cp(i).wait()
        out_cp(i).start()
        for j, deps in _BUF_DEPS.items():
            if i == max(deps):
                for d in deps:
                    out_cp(d).wait()
                in_cp(j).start()
    for i in range(n):
        if not any(i in deps for deps in _BUF_DEPS.values()):
            out_cp(i).wait()


def kernel(x):
    shape = x.shape
    x2 = x.reshape(-1, shape[-1])
    out = pl.pallas_call(
        _copy_body,
        out_shape=jax.ShapeDtypeStruct(x2.shape, x2.dtype),
        in_specs=[pl.BlockSpec(memory_space=pl.ANY)],
        out_specs=pl.BlockSpec(memory_space=pl.ANY),
        scratch_shapes=[
            pltpu.VMEM((_ARENA_ROWS, x2.shape[1]), x2.dtype),
            pltpu.SemaphoreType.DMA((len(_SCHED),)),
            pltpu.SemaphoreType.DMA((len(_SCHED),)),
        ],
    )(x2)
    return out.reshape(shape)


# speedup vs baseline: 1.0275x; 1.0275x over previous
import jax
import jax.numpy as jnp
from jax.experimental import pallas as pl
from jax.experimental.pallas import tpu as pltpu

_SCHED = (2560, 11264, 2560)
_ARENA_ROWS = 13824
_X_OFF = tuple(sum(_SCHED[:i]) for i in range(len(_SCHED)))
_BUF_OFF = (0, 2560, 0)
_BUF_DEPS = {2: (0,)}


def _copy_body(x_ref, o_ref, arena, in_sems, out_sems):
    n = len(_SCHED)

    def in_cp(i):
        return pltpu.make_async_copy(
            x_ref.at[pl.ds(_X_OFF[i], _SCHED[i])],
            arena.at[pl.ds(_BUF_OFF[i], _SCHED[i])],
            in_sems.at[i],
        )

    def out_cp(i):
        return pltpu.make_async_copy(
            arena.at[pl.ds(_BUF_OFF[i], _SCHED[i])],
            o_ref.at[pl.ds(_X_OFF[i], _SCHED[i])],
            out_sems.at[i],
        )

    for j in range(n):
        if j not in _BUF_DEPS:
            in_cp(j).start()
    for i in range(n):
        in_cp(i).wait()
        out_cp(i).start()
        for j, deps in _BUF_DEPS.items():
            if i == max(deps):
                for d in deps:
                    out_cp(d).wait()
                in_cp(j).start()
    for i in range(n):
        if not any(i in deps for deps in _BUF_DEPS.values()):
            out_cp(i).wait()


def kernel(x):
    shape = x.shape
    x2 = x.reshape(-1, shape[-1])
    out = pl.pallas_call(
        _copy_body,
        out_shape=jax.ShapeDtypeStruct(x2.shape, x2.dtype),
        in_specs=[pl.BlockSpec(memory_space=pl.ANY)],
        out_specs=pl.BlockSpec(memory_space=pl.ANY),
        scratch_shapes=[
            pltpu.VMEM((_ARENA_ROWS, x2.shape[1]), x2.dtype),
            pltpu.SemaphoreType.DMA((len(_SCHED),)),
            pltpu.SemaphoreType.DMA((len(_SCHED),)),
        ],
    )(x2)
    return out.reshape(shape)
